# baseline (speedup 1.0000x reference)
import functools

import jax
import jax.numpy as jnp
from jax import lax
from jax.experimental import pallas as pl
from jax.experimental.pallas import tpu as pltpu
from jax.experimental.pallas import tpu_sc as plsc

N = 10000
E = 320000
DIM = 128
H = 8
HID = 256
W16 = 16

NC = 2
NS = 16
EDGES_PER_SC = E // NC
EDGES_PER_TILE = EDGES_PER_SC // NS
KB = 80
CH = 5
K = KB * CH
NCHUNK = EDGES_PER_TILE // K
K2 = 80
NCHUNK2 = EDGES_PER_TILE // K2
NPAD = 10240
ROWS_PER_TILE = NPAD // NS


def _node_proj_body(x_ref, wt_ref, b_ref, wu_ref, bu_ref, wv_ref,
                    xp_ref, au_ref, av_ref):
    xp = jnp.dot(x_ref[...], wt_ref[...], preferred_element_type=jnp.float32)
    xp = xp + b_ref[...]
    xp_ref[...] = xp
    au_ref[...] = jnp.dot(xp, wu_ref[...],
                          preferred_element_type=jnp.float32) + bu_ref[...]
    av_ref[...] = jnp.dot(xp, wv_ref[...], preferred_element_type=jnp.float32)


def _node_proj(x, wt, b, wu2t, bu2, wv2t):
    blk = 1000
    grid = N // blk
    return pl.pallas_call(
        _node_proj_body,
        grid=(grid,),
        in_specs=[
            pl.BlockSpec((blk, DIM), lambda i: (i, 0)),
            pl.BlockSpec((DIM, DIM), lambda i: (0, 0)),
            pl.BlockSpec((1, DIM), lambda i: (0, 0)),
            pl.BlockSpec((DIM, W16), lambda i: (0, 0)),
            pl.BlockSpec((1, W16), lambda i: (0, 0)),
            pl.BlockSpec((DIM, W16), lambda i: (0, 0)),
        ],
        out_specs=[
            pl.BlockSpec((blk, DIM), lambda i: (i, 0)),
            pl.BlockSpec((blk, W16), lambda i: (i, 0)),
            pl.BlockSpec((blk, W16), lambda i: (i, 0)),
        ],
        out_shape=[
            jax.ShapeDtypeStruct((N, DIM), jnp.float32),
            jax.ShapeDtypeStruct((N, W16), jnp.float32),
            jax.ShapeDtypeStruct((N, W16), jnp.float32),
        ],
    )(x, wt, b, wu2t, bu2, wv2t)


def _sc_pass1(src, dst, ef2, mc, sp, au, av, zeros16):
    mesh = plsc.VectorSubcoreMesh(core_axis_name="c", subcore_axis_name="s")

    @functools.partial(
        pl.kernel,
        out_type=[
            jax.ShapeDtypeStruct((E, W16), jnp.float32),
            jax.ShapeDtypeStruct((NC, NPAD, W16), jnp.float32),
        ],
        mesh=mesh,
        compiler_params=pltpu.CompilerParams(use_tc_tiling_on_sc=False),
        scratch_types=(
            [pltpu.VMEM((K,), jnp.int32)]
            + [pltpu.VMEM((KB,), jnp.int32) for _ in range(CH)]
            + [
                pltpu.VMEM((2 * K,), jnp.float32),
                pltpu.VMEM((3, W16), jnp.float32),
                pltpu.VMEM((16, 16), jnp.int32),
                pltpu.VMEM((K, W16), jnp.float32),
                pltpu.VMEM((K, W16), jnp.float32),
                pltpu.VMEM((K, W16), jnp.float32),
                pltpu.VMEM_SHARED((NPAD, W16), jnp.float32),
                pltpu.SemaphoreType.DMA,
                pltpu.SemaphoreType.DMA,
            ]
        ),
    )
    def body(src_hbm, dst_hbm, ef_hbm, mc_hbm, sp_hbm, au_hbm, av_hbm,
             z_hbm, exp_hbm, ssum_hbm, src_v, d0, d1, d2, d3, d4,
             ef_v, mc_v, sp_v, asrc_v, advt_v, ebuf_v, ssum_sh, sem, sem2):
        dvs = [d0, d1, d2, d3, d4]
        c = lax.axis_index("c")
        s = lax.axis_index("s")
        rbase = s * ROWS_PER_TILE
        pltpu.sync_copy(z_hbm, ssum_sh.at[pl.ds(rbase, ROWS_PER_TILE)])
        pltpu.sync_copy(mc_hbm, mc_v)
        pltpu.sync_copy(sp_hbm, sp_v)
        m0t = mc_v[0, :]
        m1t = mc_v[1, :]
        ct = mc_v[2, :]
        plsc.subcore_barrier()

        tbase = c * EDGES_PER_SC + s * EDGES_PER_TILE

        def chunk(i, _):
            base = pl.multiple_of(tbase + i * K, 8)
            idxl = [pltpu.async_copy(src_hbm.at[pl.ds(base, K)], src_v, sem2)]
            for j in range(CH):
                idxl.append(pltpu.async_copy(
                    dst_hbm.at[pl.ds(base + j * KB, KB)], dvs[j], sem2))
            for d in idxl:
                d.wait()
            loads = [pltpu.async_copy(ef_hbm.at[pl.ds(2 * base, 2 * K)],
                                      ef_v, sem)]
            for j in range(CH):
                loads.append(pltpu.async_copy(
                    au_hbm.at[src_v.at[pl.ds(j * KB, KB)]],
                    asrc_v.at[pl.ds(j * KB, KB)], sem))
                loads.append(pltpu.async_copy(
                    av_hbm.at[dvs[j]],
                    advt_v.at[pl.ds(j * KB, KB)], sem))
            for ld in loads:
                ld.wait()

            def edge(g, _):
                v = ef_v[pl.ds(g * 16, 16)]
                for t in range(8):
                    e = g * 8 + t
                    f0 = v[sp_v[2 * t, :]]
                    f1 = v[sp_v[2 * t + 1, :]]
                    sc = (asrc_v[e, :] + advt_v[e, :]
                          + f0 * m0t + f1 * m1t + ct)
                    sc = jnp.where(sc >= 0.0, sc, 0.2 * sc)
                    ebuf_v[e, :] = jnp.exp(sc)
                return 0

            lax.fori_loop(0, K // 8, edge, 0)
            est = pltpu.async_copy(ebuf_v, exp_hbm.at[pl.ds(base, K)], sem2)
            for j in range(CH):
                pltpu.sync_copy(ebuf_v.at[pl.ds(j * KB, KB)],
                                ssum_sh.at[dvs[j]], add=True)
            est.wait()
            return 0

        lax.fori_loop(0, NCHUNK, chunk, 0)
        plsc.subcore_barrier()
        pltpu.sync_copy(ssum_sh.at[pl.ds(rbase, ROWS_PER_TILE)],
                        ssum_hbm.at[c, pl.ds(rbase, ROWS_PER_TILE)])

    return body(src, dst, ef2, mc, sp, au, av, zeros16)


def _combine_body(ss_ref, out_ref):
    out_ref[...] = 1.0 / (ss_ref[0] + ss_ref[1] + 1e-16)


def _combine(ssum):
    blk = 1000
    grid = N // blk
    return pl.pallas_call(
        _combine_body,
        grid=(grid,),
        in_specs=[pl.BlockSpec((NC, blk, W16), lambda i: (0, i, 0))],
        out_specs=pl.BlockSpec((blk, W16), lambda i: (i, 0)),
        out_shape=jax.ShapeDtypeStruct((N, W16), jnp.float32),
    )(ssum)


def _sc_pass2(src, dst, exps, rsum, xp, zeros128):
    mesh = plsc.VectorSubcoreMesh(core_axis_name="c", subcore_axis_name="s")
    P2 = K2 // 2

    @functools.partial(
        pl.kernel,
        out_type=jax.ShapeDtypeStruct((NC, NPAD, DIM), jnp.float32),
        mesh=mesh,
        compiler_params=pltpu.CompilerParams(use_tc_tiling_on_sc=False),
        scratch_types=(
            [pltpu.VMEM((K2,), jnp.int32) for _ in range(2)]
            + [pltpu.VMEM((K2,), jnp.int32) for _ in range(2)]
            + [pltpu.VMEM((K2, W16), jnp.float32) for _ in range(2)]
            + [pltpu.VMEM((K2, W16), jnp.float32) for _ in range(2)]
            + [pltpu.VMEM((K2, DIM), jnp.float32) for _ in range(2)]
            + [
                pltpu.VMEM_SHARED((NPAD, DIM), jnp.float32),
                pltpu.SemaphoreType.DMA,
                pltpu.SemaphoreType.DMA,
                pltpu.SemaphoreType.DMA,
                pltpu.SemaphoreType.DMA,
            ]
        ),
    )
    def body(src_hbm, dst_hbm, exp_hbm, rsum_hbm, xp_hbm, z_hbm,
             agg_hbm, s0, s1, t0, t1, e0, e1, r0, r1, x0, x1,
             agg_sh, semi0, semi1, semd0, semd1):
        sv = [s0, s1]
        tv = [t0, t1]
        ev = [e0, e1]
        rv = [r0, r1]
        xv = [x0, x1]
        semi = [semi0, semi1]
        semd = [semd0, semd1]
        c = lax.axis_index("c")
        s = lax.axis_index("s")
        rbase = s * ROWS_PER_TILE
        pltpu.sync_copy(z_hbm, agg_sh.at[pl.ds(rbase, ROWS_PER_TILE)])
        plsc.subcore_barrier()

        tbase = c * EDGES_PER_SC + s * EDGES_PER_TILE

        def cbase(i):
            return pl.multiple_of(tbase, 8) + lax.min(i, NCHUNK2 - 1) * K2

        def fire_idx(k, i):
            b = cbase(i)
            pltpu.async_copy(src_hbm.at[pl.ds(b, K2)], sv[k], semi[k])
            pltpu.async_copy(dst_hbm.at[pl.ds(b, K2)], tv[k], semi[k])

        def drain_idx(k):
            pltpu.make_async_copy(src_hbm.at[pl.ds(0, K2)], sv[k],
                                  semi[k]).wait()
            pltpu.make_async_copy(dst_hbm.at[pl.ds(0, K2)], tv[k],
                                  semi[k]).wait()

        def fire_data(k, i):
            b = cbase(i)
            pltpu.async_copy(exp_hbm.at[pl.ds(b, K2)], ev[k], semd[k])
            pltpu.async_copy(rsum_hbm.at[tv[k]], rv[k], semd[k])
            pltpu.async_copy(xp_hbm.at[sv[k]], xv[k], semd[k])

        def drain_data(k):
            pltpu.make_async_copy(exp_hbm.at[pl.ds(0, K2)], ev[k],
                                  semd[k]).wait()
            pltpu.make_async_copy(exp_hbm.at[pl.ds(0, K2)], rv[k],
                                  semd[k]).wait()
            pltpu.make_async_copy(xp_hbm.at[pl.ds(0, K2)], xv[k],
                                  semd[k]).wait()

        def compute_scatter(k):
            ebuf_v, rbuf_v, xbuf_v = ev[k], rv[k], xv[k]

            def edge(e2, _):
                for u in range(2):
                    e = e2 * 2 + u
                    p = ebuf_v[e, :] * rbuf_v[e, :]
                    for j in range(H):
                        xbuf_v[e, j * 16:(j + 1) * 16] = (
                            xbuf_v[e, j * 16:(j + 1) * 16] * p)
                return 0

            lax.fori_loop(0, P2, edge, 0)
            pltpu.sync_copy(xbuf_v, agg_sh.at[tv[k]], add=True)

        fire_idx(0, 0)
        drain_idx(0)
        fire_data(0, 0)
        fire_idx(1, 1)

        def pipe(g, _):
            i0 = 2 * g
            drain_idx(1)
            fire_data(1, i0 + 1)
            drain_data(0)
            compute_scatter(0)
            fire_idx(0, i0 + 2)
            drain_idx(0)
            drain_data(1)
            fire_data(0, i0 + 2)
            compute_scatter(1)
            fire_idx(1, i0 + 3)
            return 0

        lax.fori_loop(0, (NCHUNK2 - 1) // 2, pipe, 0)
        drain_idx(1)
        drain_data(0)
        compute_scatter(0)

        plsc.subcore_barrier()
        pltpu.sync_copy(agg_sh.at[pl.ds(rbase, ROWS_PER_TILE)],
                        agg_hbm.at[c, pl.ds(rbase, ROWS_PER_TILE)])

    return body(src, dst, exps, rsum, xp, zeros128)


def _ff_body(agg_ref, w1_ref, b1_ref, w2_ref, b2_ref, out_ref):
    a = agg_ref[0] + agg_ref[1]
    h = jnp.dot(a, w1_ref[...], preferred_element_type=jnp.float32)
    h = jax.nn.gelu(h + b1_ref[...])
    out_ref[...] = jnp.dot(h, w2_ref[...],
                           preferred_element_type=jnp.float32) + b2_ref[...]


def _ff(agg, w1t, b1, w2t, b2):
    blk = 1000
    grid = N // blk
    return pl.pallas_call(
        _ff_body,
        grid=(grid,),
        in_specs=[
            pl.BlockSpec((NC, blk, DIM), lambda i: (0, i, 0)),
            pl.BlockSpec((DIM, HID), lambda i: (0, 0)),
            pl.BlockSpec((1, HID), lambda i: (0, 0)),
            pl.BlockSpec((HID, DIM), lambda i: (0, 0)),
            pl.BlockSpec((1, DIM), lambda i: (0, 0)),
        ],
        out_specs=pl.BlockSpec((blk, DIM), lambda i: (i, 0)),
        out_shape=jax.ShapeDtypeStruct((N, DIM), jnp.float32),
    )(agg, w1t, b1, w2t, b2)


def kernel(x, edge_index, edge_feat, W_in, b_in, W_e, b_e, W_u, b_u, W_v,
           W_ae, b_ae, W_ff1, b_ff1, W_ff2, b_ff2):
    src = edge_index[0].astype(jnp.int32)
    dst = edge_index[1].astype(jnp.int32)

    wu2t = jnp.concatenate([W_u, W_u], axis=0).T
    bu2 = jnp.tile(b_u, 2).reshape(1, W16)
    wv2t = jnp.concatenate([W_v, W_v], axis=0).T
    m = W_e.T @ W_ae.T
    cvec = b_e @ W_ae.T + b_ae
    mc = jnp.stack([jnp.tile(m[0], 2), jnp.tile(m[1], 2),
                    jnp.tile(cvec, 2)])
    ef2 = edge_feat.reshape(2 * E)
    sp = jnp.broadcast_to(jnp.arange(16, dtype=jnp.int32)[:, None], (16, 16))

    zeros16 = jnp.zeros((ROWS_PER_TILE, W16), jnp.float32)
    zeros128 = jnp.zeros((ROWS_PER_TILE, DIM), jnp.float32)

    xp, au16, av16 = _node_proj(x, W_in.T, b_in.reshape(1, DIM), wu2t,
                                bu2, wv2t)
    exps, ssum = _sc_pass1(src, dst, ef2, mc, sp, au16, av16, zeros16)
    rsum = _combine(ssum)
    agg = _sc_pass2(src, dst, exps, rsum, xp, zeros128)
    return _ff(agg, W_ff1.T, b_ff1.reshape(1, HID), W_ff2.T,
               b_ff2.reshape(1, DIM))


# speedup vs baseline: 1.0959x; 1.0959x over previous
import functools

import jax
import jax.numpy as jnp
from jax import lax
from jax.experimental import pallas as pl
from jax.experimental.pallas import tpu as pltpu
from jax.experimental.pallas import tpu_sc as plsc

N = 10000
E = 320000
DIM = 128
H = 8
HID = 256
W16 = 16

NC = 2
NS = 16
EDGES_PER_SC = E // NC
EDGES_PER_TILE = EDGES_PER_SC // NS
KB = 40
CH = 5
K = KB * CH
NCHUNK = EDGES_PER_TILE // K
K2 = 80
NCHUNK2 = EDGES_PER_TILE // K2
NPAD = 10240
ROWS_PER_TILE = NPAD // NS


def _node_proj_body(x_ref, wt_ref, b_ref, wu_ref, bu_ref, wv_ref,
                    xp_ref, au_ref, av_ref):
    xp = jnp.dot(x_ref[...], wt_ref[...], preferred_element_type=jnp.float32)
    xp = xp + b_ref[...]
    xp_ref[...] = xp
    au_ref[...] = jnp.dot(xp, wu_ref[...],
                          preferred_element_type=jnp.float32) + bu_ref[...]
    av_ref[...] = jnp.dot(xp, wv_ref[...], preferred_element_type=jnp.float32)


def _node_proj(x, wt, b, wu2t, bu2, wv2t):
    blk = 1000
    grid = N // blk
    return pl.pallas_call(
        _node_proj_body,
        grid=(grid,),
        in_specs=[
            pl.BlockSpec((blk, DIM), lambda i: (i, 0)),
            pl.BlockSpec((DIM, DIM), lambda i: (0, 0)),
            pl.BlockSpec((1, DIM), lambda i: (0, 0)),
            pl.BlockSpec((DIM, W16), lambda i: (0, 0)),
            pl.BlockSpec((1, W16), lambda i: (0, 0)),
            pl.BlockSpec((DIM, W16), lambda i: (0, 0)),
        ],
        out_specs=[
            pl.BlockSpec((blk, DIM), lambda i: (i, 0)),
            pl.BlockSpec((blk, W16), lambda i: (i, 0)),
            pl.BlockSpec((blk, W16), lambda i: (i, 0)),
        ],
        out_shape=[
            jax.ShapeDtypeStruct((N, DIM), jnp.float32),
            jax.ShapeDtypeStruct((N, W16), jnp.float32),
            jax.ShapeDtypeStruct((N, W16), jnp.float32),
        ],
    )(x, wt, b, wu2t, bu2, wv2t)


def _edge_bias_body(ef_ref, m0_ref, m1_ref, c_ref, out_ref):
    ef = ef_ref[...]
    out_ref[...] = (ef[:, 0:1] * m0_ref[...] + ef[:, 1:2] * m1_ref[...]
                    + c_ref[...])


def _edge_bias(ef, m0, m1, c):
    blk = 16000
    grid = E // blk
    return pl.pallas_call(
        _edge_bias_body,
        grid=(grid,),
        in_specs=[
            pl.BlockSpec((blk, 2), lambda i: (i, 0)),
            pl.BlockSpec((1, W16), lambda i: (0, 0)),
            pl.BlockSpec((1, W16), lambda i: (0, 0)),
            pl.BlockSpec((1, W16), lambda i: (0, 0)),
        ],
        out_specs=pl.BlockSpec((blk, W16), lambda i: (i, 0)),
        out_shape=jax.ShapeDtypeStruct((E, W16), jnp.float32),
    )(ef, m0, m1, c)


def _sc_pass1(src, dst, ae, au, av, zeros16):
    mesh = plsc.VectorSubcoreMesh(core_axis_name="c", subcore_axis_name="s")

    @functools.partial(
        pl.kernel,
        out_type=[
            jax.ShapeDtypeStruct((E, W16), jnp.float32),
            jax.ShapeDtypeStruct((NC, NPAD, W16), jnp.float32),
        ],
        mesh=mesh,
        compiler_params=pltpu.CompilerParams(use_tc_tiling_on_sc=False),
        scratch_types=(
            [pltpu.VMEM((K,), jnp.int32) for _ in range(2)]
            + [pltpu.VMEM((K,), jnp.int32) for _ in range(2)]
            + [pltpu.VMEM((K, W16), jnp.float32) for _ in range(2)]
            + [pltpu.VMEM((K, W16), jnp.float32) for _ in range(2)]
            + [pltpu.VMEM((K, W16), jnp.float32) for _ in range(2)]
            + [
                pltpu.VMEM_SHARED((NPAD, W16), jnp.float32),
                pltpu.SemaphoreType.DMA,
                pltpu.SemaphoreType.DMA,
                pltpu.SemaphoreType.DMA,
                pltpu.SemaphoreType.DMA,
            ]
        ),
    )
    def body(src_hbm, dst_hbm, ae_hbm, au_hbm, av_hbm, z_hbm,
             exp_hbm, ssum_hbm, s0, s1, t0, t1, a0, a1, b0, b1, e0, e1,
             ssum_sh, semi0, semi1, semd0, semd1):
        sv = [s0, s1]
        tv = [t0, t1]
        av_ = [a0, a1]
        bv = [b0, b1]
        ev = [e0, e1]
        semi = [semi0, semi1]
        semd = [semd0, semd1]
        c = lax.axis_index("c")
        s = lax.axis_index("s")
        rbase = s * ROWS_PER_TILE
        pltpu.sync_copy(z_hbm, ssum_sh.at[pl.ds(rbase, ROWS_PER_TILE)])
        plsc.subcore_barrier()

        tbase = c * EDGES_PER_SC + s * EDGES_PER_TILE

        def cbase(i):
            return pl.multiple_of(tbase, 8) + lax.min(i, NCHUNK - 1) * K

        def fire_idx(k, i):
            b = cbase(i)
            pltpu.async_copy(src_hbm.at[pl.ds(b, K)], sv[k], semi[k])
            pltpu.async_copy(dst_hbm.at[pl.ds(b, K)], tv[k], semi[k])

        def drain_idx(k):
            pltpu.make_async_copy(src_hbm.at[pl.ds(0, K)], sv[k],
                                  semi[k]).wait()
            pltpu.make_async_copy(dst_hbm.at[pl.ds(0, K)], tv[k],
                                  semi[k]).wait()

        def fire_data(k, i):
            b = cbase(i)
            pltpu.async_copy(ae_hbm.at[pl.ds(b, K)], ev[k], semd[k])
            for j in range(CH):
                pltpu.async_copy(
                    au_hbm.at[sv[k].at[pl.ds(j * KB, KB)]],
                    av_[k].at[pl.ds(j * KB, KB)], semd[k])
                pltpu.async_copy(
                    av_hbm.at[tv[k].at[pl.ds(j * KB, KB)]],
                    bv[k].at[pl.ds(j * KB, KB)], semd[k])

        def drain_data(k):
            pltpu.make_async_copy(ae_hbm.at[pl.ds(0, K)], ev[k],
                                  semd[k]).wait()
            pltpu.make_async_copy(ae_hbm.at[pl.ds(0, K)], av_[k],
                                  semd[k]).wait()
            pltpu.make_async_copy(ae_hbm.at[pl.ds(0, K)], bv[k],
                                  semd[k]).wait()

        def compute_scatter(k, i):
            asrc_v, advt_v, ebuf_v = av_[k], bv[k], ev[k]
            b = cbase(i)

            def edge(e4, _):
                for u in range(4):
                    e = e4 * 4 + u
                    sc = asrc_v[e, :] + advt_v[e, :] + ebuf_v[e, :]
                    sc = jnp.where(sc >= 0.0, sc, 0.2 * sc)
                    ebuf_v[e, :] = jnp.exp(sc)
                return 0

            lax.fori_loop(0, K // 4, edge, 0)
            est = pltpu.async_copy(ebuf_v, exp_hbm.at[pl.ds(b, K)], semi[k])
            pltpu.sync_copy(ebuf_v, ssum_sh.at[tv[k]], add=True)
            est.wait()

        fire_idx(0, 0)
        drain_idx(0)
        fire_data(0, 0)
        fire_idx(1, 1)

        def pipe(g, _):
            i0 = 2 * g
            drain_idx(1)
            fire_data(1, i0 + 1)
            drain_data(0)
            compute_scatter(0, i0)
            fire_idx(0, i0 + 2)
            drain_idx(0)
            drain_data(1)
            fire_data(0, i0 + 2)
            compute_scatter(1, i0 + 1)
            fire_idx(1, i0 + 3)
            return 0

        lax.fori_loop(0, NCHUNK // 2 - 1, pipe, 0)
        drain_idx(1)
        fire_data(1, NCHUNK - 1)
        drain_data(0)
        compute_scatter(0, NCHUNK - 2)
        drain_data(1)
        compute_scatter(1, NCHUNK - 1)

        plsc.subcore_barrier()
        pltpu.sync_copy(ssum_sh.at[pl.ds(rbase, ROWS_PER_TILE)],
                        ssum_hbm.at[c, pl.ds(rbase, ROWS_PER_TILE)])

    return body(src, dst, ae, au, av, zeros16)


def _combine_body(ss_ref, out_ref):
    out_ref[...] = 1.0 / (ss_ref[0] + ss_ref[1] + 1e-16)


def _combine(ssum):
    blk = 1000
    grid = N // blk
    return pl.pallas_call(
        _combine_body,
        grid=(grid,),
        in_specs=[pl.BlockSpec((NC, blk, W16), lambda i: (0, i, 0))],
        out_specs=pl.BlockSpec((blk, W16), lambda i: (i, 0)),
        out_shape=jax.ShapeDtypeStruct((N, W16), jnp.float32),
    )(ssum)


def _sc_pass2(src, dst, exps, rsum, xp, zeros128):
    mesh = plsc.VectorSubcoreMesh(core_axis_name="c", subcore_axis_name="s")
    P2 = K2 // 2

    @functools.partial(
        pl.kernel,
        out_type=jax.ShapeDtypeStruct((NC, NPAD, DIM), jnp.float32),
        mesh=mesh,
        compiler_params=pltpu.CompilerParams(use_tc_tiling_on_sc=False),
        scratch_types=(
            [pltpu.VMEM((K2,), jnp.int32) for _ in range(2)]
            + [pltpu.VMEM((K2,), jnp.int32) for _ in range(2)]
            + [pltpu.VMEM((K2, W16), jnp.float32) for _ in range(2)]
            + [pltpu.VMEM((K2, W16), jnp.float32) for _ in range(2)]
            + [pltpu.VMEM((K2, DIM), jnp.float32) for _ in range(2)]
            + [
                pltpu.VMEM_SHARED((NPAD, DIM), jnp.float32),
                pltpu.SemaphoreType.DMA,
                pltpu.SemaphoreType.DMA,
                pltpu.SemaphoreType.DMA,
                pltpu.SemaphoreType.DMA,
            ]
        ),
    )
    def body(src_hbm, dst_hbm, exp_hbm, rsum_hbm, xp_hbm, z_hbm,
             agg_hbm, s0, s1, t0, t1, e0, e1, r0, r1, x0, x1,
             agg_sh, semi0, semi1, semd0, semd1):
        sv = [s0, s1]
        tv = [t0, t1]
        ev = [e0, e1]
        rv = [r0, r1]
        xv = [x0, x1]
        semi = [semi0, semi1]
        semd = [semd0, semd1]
        c = lax.axis_index("c")
        s = lax.axis_index("s")
        rbase = s * ROWS_PER_TILE
        pltpu.sync_copy(z_hbm, agg_sh.at[pl.ds(rbase, ROWS_PER_TILE)])
        plsc.subcore_barrier()

        tbase = c * EDGES_PER_SC + s * EDGES_PER_TILE

        def cbase(i):
            return pl.multiple_of(tbase, 8) + lax.min(i, NCHUNK2 - 1) * K2

        def fire_idx(k, i):
            b = cbase(i)
            pltpu.async_copy(src_hbm.at[pl.ds(b, K2)], sv[k], semi[k])
            pltpu.async_copy(dst_hbm.at[pl.ds(b, K2)], tv[k], semi[k])

        def drain_idx(k):
            pltpu.make_async_copy(src_hbm.at[pl.ds(0, K2)], sv[k],
                                  semi[k]).wait()
            pltpu.make_async_copy(dst_hbm.at[pl.ds(0, K2)], tv[k],
                                  semi[k]).wait()

        def fire_data(k, i):
            b = cbase(i)
            pltpu.async_copy(exp_hbm.at[pl.ds(b, K2)], ev[k], semd[k])
            pltpu.async_copy(rsum_hbm.at[tv[k]], rv[k], semd[k])
            pltpu.async_copy(xp_hbm.at[sv[k]], xv[k], semd[k])

        def drain_data(k):
            pltpu.make_async_copy(exp_hbm.at[pl.ds(0, K2)], ev[k],
                                  semd[k]).wait()
            pltpu.make_async_copy(exp_hbm.at[pl.ds(0, K2)], rv[k],
                                  semd[k]).wait()
            pltpu.make_async_copy(xp_hbm.at[pl.ds(0, K2)], xv[k],
                                  semd[k]).wait()

        def compute_scatter(k):
            ebuf_v, rbuf_v, xbuf_v = ev[k], rv[k], xv[k]

            def edge(e2, _):
                for u in range(2):
                    e = e2 * 2 + u
                    p = ebuf_v[e, :] * rbuf_v[e, :]
                    for j in range(H):
                        xbuf_v[e, j * 16:(j + 1) * 16] = (
                            xbuf_v[e, j * 16:(j + 1) * 16] * p)
                return 0

            lax.fori_loop(0, P2, edge, 0)
            pltpu.sync_copy(xbuf_v, agg_sh.at[tv[k]], add=True)

        fire_idx(0, 0)
        drain_idx(0)
        fire_data(0, 0)
        fire_idx(1, 1)

        def pipe(g, _):
            i0 = 2 * g
            drain_idx(1)
            fire_data(1, i0 + 1)
            drain_data(0)
            compute_scatter(0)
            fire_idx(0, i0 + 2)
            drain_idx(0)
            drain_data(1)
            fire_data(0, i0 + 2)
            compute_scatter(1)
            fire_idx(1, i0 + 3)
            return 0

        lax.fori_loop(0, (NCHUNK2 - 1) // 2, pipe, 0)
        drain_idx(1)
        drain_data(0)
        compute_scatter(0)

        plsc.subcore_barrier()
        pltpu.sync_copy(agg_sh.at[pl.ds(rbase, ROWS_PER_TILE)],
                        agg_hbm.at[c, pl.ds(rbase, ROWS_PER_TILE)])

    return body(src, dst, exps, rsum, xp, zeros128)


def _ff_body(agg_ref, w1_ref, b1_ref, w2_ref, b2_ref, out_ref):
    a = agg_ref[0] + agg_ref[1]
    h = jnp.dot(a, w1_ref[...], preferred_element_type=jnp.float32)
    h = jax.nn.gelu(h + b1_ref[...])
    out_ref[...] = jnp.dot(h, w2_ref[...],
                           preferred_element_type=jnp.float32) + b2_ref[...]


def _ff(agg, w1t, b1, w2t, b2):
    blk = 1000
    grid = N // blk
    return pl.pallas_call(
        _ff_body,
        grid=(grid,),
        in_specs=[
            pl.BlockSpec((NC, blk, DIM), lambda i: (0, i, 0)),
            pl.BlockSpec((DIM, HID), lambda i: (0, 0)),
            pl.BlockSpec((1, HID), lambda i: (0, 0)),
            pl.BlockSpec((HID, DIM), lambda i: (0, 0)),
            pl.BlockSpec((1, DIM), lambda i: (0, 0)),
        ],
        out_specs=pl.BlockSpec((blk, DIM), lambda i: (i, 0)),
        out_shape=jax.ShapeDtypeStruct((N, DIM), jnp.float32),
    )(agg, w1t, b1, w2t, b2)


def kernel(x, edge_index, edge_feat, W_in, b_in, W_e, b_e, W_u, b_u, W_v,
           W_ae, b_ae, W_ff1, b_ff1, W_ff2, b_ff2):
    src = edge_index[0].astype(jnp.int32)
    dst = edge_index[1].astype(jnp.int32)

    wu2t = jnp.concatenate([W_u, W_u], axis=0).T
    bu2 = jnp.tile(b_u, 2).reshape(1, W16)
    wv2t = jnp.concatenate([W_v, W_v], axis=0).T
    m = W_e.T @ W_ae.T
    cvec = b_e @ W_ae.T + b_ae
    m0 = jnp.tile(m[0], 2).reshape(1, W16)
    m1 = jnp.tile(m[1], 2).reshape(1, W16)
    c16 = jnp.tile(cvec, 2).reshape(1, W16)

    zeros16 = jnp.zeros((ROWS_PER_TILE, W16), jnp.float32)
    zeros128 = jnp.zeros((ROWS_PER_TILE, DIM), jnp.float32)

    xp, au16, av16 = _node_proj(x, W_in.T, b_in.reshape(1, DIM), wu2t,
                                bu2, wv2t)
    ae16 = _edge_bias(edge_feat, m0, m1, c16)
    exps, ssum = _sc_pass1(src, dst, ae16, au16, av16, zeros16)
    rsum = _combine(ssum)
    agg = _sc_pass2(src, dst, exps, rsum, xp, zeros128)
    return _ff(agg, W_ff1.T, b_ff1.reshape(1, HID), W_ff2.T,
               b_ff2.reshape(1, DIM))


# speedup vs baseline: 1.2084x; 1.1027x over previous
import functools

import jax
import jax.numpy as jnp
from jax import lax
from jax.experimental import pallas as pl
from jax.experimental.pallas import tpu as pltpu
from jax.experimental.pallas import tpu_sc as plsc

N = 10000
E = 320000
DIM = 128
H = 8
HID = 256
W16 = 16

NC = 2
NS = 16
EDGES_PER_SC = E // NC
EDGES_PER_TILE = EDGES_PER_SC // NS
KB = 40
CH = 5
K = KB * CH
NCHUNK = EDGES_PER_TILE // K
K2 = 80
NCHUNK2 = EDGES_PER_TILE // K2
NPAD = 10240
ROWS_PER_TILE = NPAD // NS


def _node_proj_body(x_ref, wt_ref, b_ref, wu_ref, bu_ref, wv_ref,
                    xp_ref, au_ref, av_ref):
    xp = jnp.dot(x_ref[...], wt_ref[...], preferred_element_type=jnp.float32)
    xp = xp + b_ref[...]
    xp_ref[...] = xp
    au_ref[...] = jnp.dot(xp, wu_ref[...],
                          preferred_element_type=jnp.float32) + bu_ref[...]
    av_ref[...] = jnp.dot(xp, wv_ref[...], preferred_element_type=jnp.float32)


def _node_proj(x, wt, b, wu2t, bu2, wv2t):
    blk = 1000
    grid = N // blk
    return pl.pallas_call(
        _node_proj_body,
        grid=(grid,),
        in_specs=[
            pl.BlockSpec((blk, DIM), lambda i: (i, 0)),
            pl.BlockSpec((DIM, DIM), lambda i: (0, 0)),
            pl.BlockSpec((1, DIM), lambda i: (0, 0)),
            pl.BlockSpec((DIM, W16), lambda i: (0, 0)),
            pl.BlockSpec((1, W16), lambda i: (0, 0)),
            pl.BlockSpec((DIM, W16), lambda i: (0, 0)),
        ],
        out_specs=[
            pl.BlockSpec((blk, DIM), lambda i: (i, 0)),
            pl.BlockSpec((blk, W16), lambda i: (i, 0)),
            pl.BlockSpec((blk, W16), lambda i: (i, 0)),
        ],
        out_shape=[
            jax.ShapeDtypeStruct((N, DIM), jnp.float32),
            jax.ShapeDtypeStruct((N, W16), jnp.float32),
            jax.ShapeDtypeStruct((N, W16), jnp.float32),
        ],
    )(x, wt, b, wu2t, bu2, wv2t)


def _edge_bias_body(ef_ref, w_ref, c_ref, out_ref):
    h = jnp.dot(ef_ref[...], w_ref[...], preferred_element_type=jnp.float32)
    h = h + c_ref[...]
    out_ref[...] = h.reshape(h.shape[0] * 8, 128)


def _edge_bias(ef128, wbig, cbig):
    blk = 1000
    grid = (2 * E // 128) // blk
    return pl.pallas_call(
        _edge_bias_body,
        grid=(grid,),
        in_specs=[
            pl.BlockSpec((blk, 128), lambda i: (i, 0)),
            pl.BlockSpec((128, 1024), lambda i: (0, 0)),
            pl.BlockSpec((1, 1024), lambda i: (0, 0)),
        ],
        out_specs=pl.BlockSpec((blk * 8, 128), lambda i: (i, 0)),
        out_shape=jax.ShapeDtypeStruct((E * W16 // 128, 128), jnp.float32),
    )(ef128, wbig, cbig)


def _sc_pass1(src, dst, ae, au, av, zeros16):
    mesh = plsc.VectorSubcoreMesh(core_axis_name="c", subcore_axis_name="s")

    @functools.partial(
        pl.kernel,
        out_type=[
            jax.ShapeDtypeStruct((E, W16), jnp.float32),
            jax.ShapeDtypeStruct((NC, NPAD, W16), jnp.float32),
        ],
        mesh=mesh,
        compiler_params=pltpu.CompilerParams(use_tc_tiling_on_sc=False),
        scratch_types=(
            [pltpu.VMEM((K,), jnp.int32) for _ in range(2)]
            + [pltpu.VMEM((K,), jnp.int32) for _ in range(2)]
            + [pltpu.VMEM((K, W16), jnp.float32) for _ in range(2)]
            + [pltpu.VMEM((K, W16), jnp.float32) for _ in range(2)]
            + [pltpu.VMEM((K, W16), jnp.float32) for _ in range(2)]
            + [
                pltpu.VMEM_SHARED((NPAD, W16), jnp.float32),
                pltpu.SemaphoreType.DMA,
                pltpu.SemaphoreType.DMA,
                pltpu.SemaphoreType.DMA,
                pltpu.SemaphoreType.DMA,
            ]
        ),
    )
    def body(src_hbm, dst_hbm, ae_hbm, au_hbm, av_hbm, z_hbm,
             exp_hbm, ssum_hbm, s0, s1, t0, t1, a0, a1, b0, b1, e0, e1,
             ssum_sh, semi0, semi1, semd0, semd1):
        sv = [s0, s1]
        tv = [t0, t1]
        av_ = [a0, a1]
        bv = [b0, b1]
        ev = [e0, e1]
        semi = [semi0, semi1]
        semd = [semd0, semd1]
        c = lax.axis_index("c")
        s = lax.axis_index("s")
        rbase = s * ROWS_PER_TILE
        pltpu.sync_copy(z_hbm, ssum_sh.at[pl.ds(rbase, ROWS_PER_TILE)])
        plsc.subcore_barrier()

        tbase = c * EDGES_PER_SC + s * EDGES_PER_TILE

        def cbase(i):
            return pl.multiple_of(tbase, 8) + lax.min(i, NCHUNK - 1) * K

        def fire_idx(k, i):
            b = cbase(i)
            pltpu.async_copy(src_hbm.at[pl.ds(b, K)], sv[k], semi[k])
            pltpu.async_copy(dst_hbm.at[pl.ds(b, K)], tv[k], semi[k])

        def drain_idx(k):
            pltpu.make_async_copy(src_hbm.at[pl.ds(0, K)], sv[k],
                                  semi[k]).wait()
            pltpu.make_async_copy(dst_hbm.at[pl.ds(0, K)], tv[k],
                                  semi[k]).wait()

        def fire_data(k, i):
            b = cbase(i)
            pltpu.async_copy(ae_hbm.at[pl.ds(b, K)], ev[k], semd[k])
            for j in range(CH):
                pltpu.async_copy(
                    au_hbm.at[sv[k].at[pl.ds(j * KB, KB)]],
                    av_[k].at[pl.ds(j * KB, KB)], semd[k])
                pltpu.async_copy(
                    av_hbm.at[tv[k].at[pl.ds(j * KB, KB)]],
                    bv[k].at[pl.ds(j * KB, KB)], semd[k])

        def drain_data(k):
            pltpu.make_async_copy(ae_hbm.at[pl.ds(0, K)], ev[k],
                                  semd[k]).wait()
            pltpu.make_async_copy(ae_hbm.at[pl.ds(0, K)], av_[k],
                                  semd[k]).wait()
            pltpu.make_async_copy(ae_hbm.at[pl.ds(0, K)], bv[k],
                                  semd[k]).wait()

        def compute_scatter(k, i):
            asrc_v, advt_v, ebuf_v = av_[k], bv[k], ev[k]
            b = cbase(i)

            def edge(e4, _):
                for u in range(4):
                    e = e4 * 4 + u
                    sc = asrc_v[e, :] + advt_v[e, :] + ebuf_v[e, :]
                    sc = jnp.where(sc >= 0.0, sc, 0.2 * sc)
                    ebuf_v[e, :] = jnp.exp(sc)
                return 0

            lax.fori_loop(0, K // 4, edge, 0)
            est = pltpu.async_copy(ebuf_v, exp_hbm.at[pl.ds(b, K)], semi[k])
            pltpu.sync_copy(ebuf_v, ssum_sh.at[tv[k]], add=True)
            est.wait()

        fire_idx(0, 0)
        drain_idx(0)
        fire_data(0, 0)
        fire_idx(1, 1)

        def pipe(g, _):
            i0 = 2 * g
            drain_idx(1)
            fire_data(1, i0 + 1)
            drain_data(0)
            compute_scatter(0, i0)
            fire_idx(0, i0 + 2)
            drain_idx(0)
            drain_data(1)
            fire_data(0, i0 + 2)
            compute_scatter(1, i0 + 1)
            fire_idx(1, i0 + 3)
            return 0

        lax.fori_loop(0, NCHUNK // 2 - 1, pipe, 0)
        drain_idx(1)
        fire_data(1, NCHUNK - 1)
        drain_data(0)
        compute_scatter(0, NCHUNK - 2)
        drain_data(1)
        compute_scatter(1, NCHUNK - 1)

        plsc.subcore_barrier()
        pltpu.sync_copy(ssum_sh.at[pl.ds(rbase, ROWS_PER_TILE)],
                        ssum_hbm.at[c, pl.ds(rbase, ROWS_PER_TILE)])

    return body(src, dst, ae, au, av, zeros16)


def _combine_body(ss_ref, out_ref):
    out_ref[...] = 1.0 / (ss_ref[0] + ss_ref[1] + 1e-16)


def _combine(ssum):
    blk = 1000
    grid = N // blk
    return pl.pallas_call(
        _combine_body,
        grid=(grid,),
        in_specs=[pl.BlockSpec((NC, blk, W16), lambda i: (0, i, 0))],
        out_specs=pl.BlockSpec((blk, W16), lambda i: (i, 0)),
        out_shape=jax.ShapeDtypeStruct((N, W16), jnp.float32),
    )(ssum)


def _sc_pass2(src, dst, exps, rsum, xp, zeros128):
    mesh = plsc.VectorSubcoreMesh(core_axis_name="c", subcore_axis_name="s")
    P2 = K2 // 2

    @functools.partial(
        pl.kernel,
        out_type=jax.ShapeDtypeStruct((NC, NPAD, DIM), jnp.float32),
        mesh=mesh,
        compiler_params=pltpu.CompilerParams(use_tc_tiling_on_sc=False),
        scratch_types=(
            [pltpu.VMEM((K2,), jnp.int32) for _ in range(2)]
            + [pltpu.VMEM((K2,), jnp.int32) for _ in range(2)]
            + [pltpu.VMEM((K2, W16), jnp.float32) for _ in range(2)]
            + [pltpu.VMEM((K2, W16), jnp.float32) for _ in range(2)]
            + [pltpu.VMEM((K2, DIM), jnp.float32) for _ in range(2)]
            + [
                pltpu.VMEM_SHARED((NPAD, DIM), jnp.float32),
                pltpu.SemaphoreType.DMA,
                pltpu.SemaphoreType.DMA,
                pltpu.SemaphoreType.DMA,
                pltpu.SemaphoreType.DMA,
            ]
        ),
    )
    def body(src_hbm, dst_hbm, exp_hbm, rsum_hbm, xp_hbm, z_hbm,
             agg_hbm, s0, s1, t0, t1, e0, e1, r0, r1, x0, x1,
             agg_sh, semi0, semi1, semd0, semd1):
        sv = [s0, s1]
        tv = [t0, t1]
        ev = [e0, e1]
        rv = [r0, r1]
        xv = [x0, x1]
        semi = [semi0, semi1]
        semd = [semd0, semd1]
        c = lax.axis_index("c")
        s = lax.axis_index("s")
        rbase = s * ROWS_PER_TILE
        pltpu.sync_copy(z_hbm, agg_sh.at[pl.ds(rbase, ROWS_PER_TILE)])
        plsc.subcore_barrier()

        tbase = c * EDGES_PER_SC + s * EDGES_PER_TILE

        def cbase(i):
            return pl.multiple_of(tbase, 8) + lax.min(i, NCHUNK2 - 1) * K2

        def fire_idx(k, i):
            b = cbase(i)
            pltpu.async_copy(src_hbm.at[pl.ds(b, K2)], sv[k], semi[k])
            pltpu.async_copy(dst_hbm.at[pl.ds(b, K2)], tv[k], semi[k])

        def drain_idx(k):
            pltpu.make_async_copy(src_hbm.at[pl.ds(0, K2)], sv[k],
                                  semi[k]).wait()
            pltpu.make_async_copy(dst_hbm.at[pl.ds(0, K2)], tv[k],
                                  semi[k]).wait()

        def fire_data(k, i):
            b = cbase(i)
            pltpu.async_copy(exp_hbm.at[pl.ds(b, K2)], ev[k], semd[k])
            pltpu.async_copy(rsum_hbm.at[tv[k]], rv[k], semd[k])
            pltpu.async_copy(xp_hbm.at[sv[k]], xv[k], semd[k])

        def drain_data(k):
            pltpu.make_async_copy(exp_hbm.at[pl.ds(0, K2)], ev[k],
                                  semd[k]).wait()
            pltpu.make_async_copy(exp_hbm.at[pl.ds(0, K2)], rv[k],
                                  semd[k]).wait()
            pltpu.make_async_copy(xp_hbm.at[pl.ds(0, K2)], xv[k],
                                  semd[k]).wait()

        def compute_scatter(k):
            ebuf_v, rbuf_v, xbuf_v = ev[k], rv[k], xv[k]

            def edge(e2, _):
                for u in range(2):
                    e = e2 * 2 + u
                    p = ebuf_v[e, :] * rbuf_v[e, :]
                    for j in range(H):
                        xbuf_v[e, j * 16:(j + 1) * 16] = (
                            xbuf_v[e, j * 16:(j + 1) * 16] * p)
                return 0

            lax.fori_loop(0, P2, edge, 0)
            pltpu.sync_copy(xbuf_v, agg_sh.at[tv[k]], add=True)

        fire_idx(0, 0)
        drain_idx(0)
        fire_data(0, 0)
        fire_idx(1, 1)

        def pipe(g, _):
            i0 = 2 * g
            drain_idx(1)
            fire_data(1, i0 + 1)
            drain_data(0)
            compute_scatter(0)
            fire_idx(0, i0 + 2)
            drain_idx(0)
            drain_data(1)
            fire_data(0, i0 + 2)
            compute_scatter(1)
            fire_idx(1, i0 + 3)
            return 0

        lax.fori_loop(0, (NCHUNK2 - 1) // 2, pipe, 0)
        drain_idx(1)
        drain_data(0)
        compute_scatter(0)

        plsc.subcore_barrier()
        pltpu.sync_copy(agg_sh.at[pl.ds(rbase, ROWS_PER_TILE)],
                        agg_hbm.at[c, pl.ds(rbase, ROWS_PER_TILE)])

    return body(src, dst, exps, rsum, xp, zeros128)


def _ff_body(agg_ref, w1_ref, b1_ref, w2_ref, b2_ref, out_ref):
    a = agg_ref[0] + agg_ref[1]
    h = jnp.dot(a, w1_ref[...], preferred_element_type=jnp.float32)
    h = jax.nn.gelu(h + b1_ref[...])
    out_ref[...] = jnp.dot(h, w2_ref[...],
                           preferred_element_type=jnp.float32) + b2_ref[...]


def _ff(agg, w1t, b1, w2t, b2):
    blk = 1000
    grid = N // blk
    return pl.pallas_call(
        _ff_body,
        grid=(grid,),
        in_specs=[
            pl.BlockSpec((NC, blk, DIM), lambda i: (0, i, 0)),
            pl.BlockSpec((DIM, HID), lambda i: (0, 0)),
            pl.BlockSpec((1, HID), lambda i: (0, 0)),
            pl.BlockSpec((HID, DIM), lambda i: (0, 0)),
            pl.BlockSpec((1, DIM), lambda i: (0, 0)),
        ],
        out_specs=pl.BlockSpec((blk, DIM), lambda i: (i, 0)),
        out_shape=jax.ShapeDtypeStruct((N, DIM), jnp.float32),
    )(agg, w1t, b1, w2t, b2)


def kernel(x, edge_index, edge_feat, W_in, b_in, W_e, b_e, W_u, b_u, W_v,
           W_ae, b_ae, W_ff1, b_ff1, W_ff2, b_ff2):
    src = edge_index[0].astype(jnp.int32)
    dst = edge_index[1].astype(jnp.int32)

    wu2t = jnp.concatenate([W_u, W_u], axis=0).T
    bu2 = jnp.tile(b_u, 2).reshape(1, W16)
    wv2t = jnp.concatenate([W_v, W_v], axis=0).T
    m = W_e.T @ W_ae.T
    cvec = b_e @ W_ae.T + b_ae
    mm0 = jnp.tile(m[0], 2)
    mm1 = jnp.tile(m[1], 2)
    cc = jnp.arange(128)
    pp = jnp.arange(1024)
    pmod = pp % W16
    mask = (cc[:, None] // 2) == (pp[None, :] // W16)
    vals = jnp.where((cc[:, None] % 2) == 0, mm0[pmod][None, :],
                     mm1[pmod][None, :])
    wbig = jnp.where(mask, vals, 0.0).astype(jnp.float32)
    cbig = jnp.tile(jnp.tile(cvec, 2), 64).reshape(1, 1024)
    ef128 = edge_feat.reshape(2 * E // 128, 128)

    zeros16 = jnp.zeros((ROWS_PER_TILE, W16), jnp.float32)
    zeros128 = jnp.zeros((ROWS_PER_TILE, DIM), jnp.float32)

    xp, au16, av16 = _node_proj(x, W_in.T, b_in.reshape(1, DIM), wu2t,
                                bu2, wv2t)
    ae16 = _edge_bias(ef128, wbig, cbig).reshape(E, W16)
    exps, ssum = _sc_pass1(src, dst, ae16, au16, av16, zeros16)
    rsum = _combine(ssum)
    agg = _sc_pass2(src, dst, exps, rsum, xp, zeros128)
    return _ff(agg, W_ff1.T, b_ff1.reshape(1, HID), W_ff2.T,
               b_ff2.reshape(1, DIM))
